# baseline (speedup 1.0000x reference)
import jax
import jax.numpy as jnp
from jax import lax
from jax.experimental import pallas as pl
from jax.experimental.pallas import tpu as pltpu
from jax.experimental.pallas import tpu_sc as plsc


def kernel(x, edge_index, out_index, W_in, b_in, W_out, b_out,
              p0_Wrel, p0_brel, p0_Wroot, p1_Wrel, p1_brel, p1_Wroot):
            N = x.shape[0]
            E = edge_index.shape[1]
            src, dst = edge_index[0], edge_index[1]
            osrc, odst = out_index[0], out_index[1]
            K1, K2 = 5000, 2500
            emask = jnp.ones((E,), x.dtype)
            xw = x @ W_in
            deg = jnp.zeros((N,), x.dtype).at[dst].add(emask) + 1.0
            dinv = lax.rsqrt(deg)
            p = jnp.argsort(dst, stable=True)
            srcp, dstp = src[p], dst[p]
            upd = xw[srcp] * (dinv[srcp] * dinv[dstp])[:, None]
            s1 = jnp.zeros_like(xw).at[dstp].add(upd, indices_are_sorted=True)
            h1 = jax.nn.relu(s1 + xw * (dinv * dinv)[:, None] + b_in)
            agg1 = jnp.zeros_like(h1).at[dstp].add(h1[srcp], indices_are_sorted=True)
            score1 = (agg1 @ p0_Wrel + p0_brel + h1 @ p0_Wroot)[:, 0]
            vals1, perm1 = lax.top_k(score1, K1)
            h2 = h1[perm1] * jnp.tanh(vals1)[:, None]
            mapping = jnp.full((N,), -1, jnp.int32).at[perm1].set(
                jnp.arange(K1, dtype=jnp.int32))
            ns = mapping[src]
            nd = mapping[dst]
            valid = (ns >= 0) & (nd >= 0)
            ns = jnp.where(valid, ns, 0)
            nd = jnp.where(valid, nd, 0)
            em2 = valid.astype(x.dtype)
            p2 = jnp.argsort(nd, stable=True)
            agg2 = jnp.zeros((K1, 16), x.dtype).at[nd[p2]].add(
                h2[ns[p2]] * em2[p2][:, None], indices_are_sorted=True)
            score2 = (agg2 @ p1_Wrel + p1_brel + h2 @ p1_Wroot)[:, 0]
            vals2, perm2 = lax.top_k(score2, K2)
            h3 = h2[perm2] * jnp.tanh(vals2)[:, None]
            dego = jnp.zeros((K2,), x.dtype).at[odst].add(
                jnp.ones((osrc.shape[0],), x.dtype)) + 1.0
            dinvo = lax.rsqrt(dego)
            p3 = jnp.argsort(odst, stable=True)
            g3 = h3 * dinvo[:, None]
            s3 = jnp.zeros_like(g3).at[odst[p3]].add(g3[osrc[p3]],
                                                     indices_are_sorted=True)
            return jax.nn.relu((dinvo[:, None] * (s3 + g3)) @ W_out + b_out)


# speedup vs baseline: 1.0514x; 1.0514x over previous
import functools

import jax
import jax.numpy as jnp
from jax import lax
from jax.experimental import pallas as pl
from jax.experimental.pallas import tpu as pltpu
from jax.experimental.pallas import tpu_sc as plsc

_E2, _B0 = 321536, 10240
_E2o, _Bo = 81920, 2560
_NCH0, _NCHo = 157, 40


def _hist_tile(s, keys_hbm, out_hbm, NCH, B, r):
    SL = B // 16
    keys2d, ones_v, slice_v = r["keys2d"], r["ones_v"], r["slice_v"]
    acc_sh = r["acc_sh"]
    fz = jnp.zeros((16,), jnp.float32)
    tbase = pl.multiple_of(s * (NCH * 128), 8)

    def _zo(i, _):
        ones_v[pl.ds(pl.multiple_of(i * 16, 16), 16)] = fz + 1.0
        return 0

    lax.fori_loop(0, 8, _zo, 0)

    def _zs(i, _):
        slice_v[pl.ds(pl.multiple_of(i * 16, 16), 16)] = fz
        return 0

    lax.fori_loop(0, SL // 16, _zs, 0)
    lo = pl.multiple_of(s * SL, 8)
    pltpu.sync_copy(slice_v.at[pl.ds(0, SL)], acc_sh.at[pl.ds(lo, SL)])
    plsc.subcore_barrier()

    for k in range(NCH):
        pltpu.sync_copy(keys_hbm.at[pl.ds(tbase + k * 128, 128)],
                        keys2d.at[k % 2])
        pltpu.sync_copy(ones_v, acc_sh.at[keys2d.at[k % 2]], add=True)
    plsc.subcore_barrier()

    pltpu.sync_copy(acc_sh.at[pl.ds(lo, SL)], slice_v.at[pl.ds(0, SL)])
    pltpu.sync_copy(slice_v.at[pl.ds(0, SL)], out_hbm.at[pl.ds(lo, SL)])


@functools.partial(
    pl.kernel,
    out_type=(
        jax.ShapeDtypeStruct((_B0,), jnp.float32),
        jax.ShapeDtypeStruct((_Bo,), jnp.float32),
    ),
    mesh=plsc.VectorSubcoreMesh(core_axis_name="c", subcore_axis_name="s"),
    scratch_types=dict(
        keys2d=pltpu.VMEM((2, 128), jnp.int32),
        ones_v=pltpu.VMEM((128,), jnp.float32),
        slice_v=pltpu.VMEM((_B0 // 16,), jnp.float32),
        acc_sh=pltpu.VMEM_SHARED((_B0,), jnp.float32),
    ),
)
def _deg_kernel(dst_f, odst_f, deg_out, dego_out, **r):
    c = lax.axis_index("c")
    s = lax.axis_index("s")

    @pl.when(c == 0)
    def _():
        _hist_tile(s, dst_f, deg_out, _NCH0, _B0, r)

    @pl.when(c == 1)
    def _():
        _hist_tile(s, odst_f, dego_out, _NCHo, _Bo, r)


@functools.partial(
    pl.kernel,
    out_type=jax.ShapeDtypeStruct((2, _Bo, 16), jnp.float32),
    mesh=plsc.VectorSubcoreMesh(core_axis_name="c", subcore_axis_name="s"),
    scratch_types=dict(
        upd_v=pltpu.VMEM((128, 16), jnp.float32),
        dst_v=pltpu.VMEM(((_NCHo // 2) * 128,), jnp.int32),
        idx2d=pltpu.VMEM((1, 128), jnp.int32),
        slice_v=pltpu.VMEM((40, 16), jnp.float32),
        acc_sh=pltpu.VMEM_SHARED((648, 16), jnp.float32),
    ),
)
def _fin_kernel(upd2, odst_f, out, *, upd_v, dst_v, idx2d, slice_v, acc_sh):
    NCHT = _NCHo // 2
    NPASS, PR = 4, 640
    c = lax.axis_index("c")
    s = lax.axis_index("s")
    fz = jnp.zeros((16,), jnp.float32)
    SLR = PR // 16

    w = c * 16 + s
    ebase = pl.multiple_of(w * (NCHT * 128), 8)
    pltpu.sync_copy(odst_f.at[pl.ds(ebase, NCHT * 128)],
                    dst_v.at[pl.ds(0, NCHT * 128)])

    def _zs(i, _):
        slice_v[i, pl.ds(0, 16)] = fz
        return 0

    lax.fori_loop(0, SLR, _zs, 0)
    lo = pl.multiple_of(s * SLR, 8)

    for pp in range(NPASS):
        plo = pp * PR
        pltpu.sync_copy(slice_v.at[pl.ds(0, SLR)], acc_sh.at[pl.ds(lo, SLR)])
        plsc.subcore_barrier()
        for k in range(NCHT):
            for g in range(8):
                o = pl.multiple_of(g * 16, 16)
                iv = dst_v[pl.ds(pl.multiple_of(k * 128 + g * 16, 16), 16)]
                inr = (iv >= plo) & (iv < plo + PR)
                idx2d[0, pl.ds(o, 16)] = jnp.where(inr, iv - plo,
                                                   jnp.int32(PR))
            pltpu.sync_copy(upd2.at[pl.ds(ebase + k * 128, 128)], upd_v)
            pltpu.sync_copy(upd_v, acc_sh.at[idx2d.at[0]], add=True)
        plsc.subcore_barrier()
        pltpu.sync_copy(acc_sh.at[pl.ds(lo, SLR)], slice_v.at[pl.ds(0, SLR)])
        pltpu.sync_copy(slice_v.at[pl.ds(0, SLR)],
                        out.at[c, pl.ds(pl.multiple_of(plo + s * SLR, 8), SLR)])
        plsc.subcore_barrier()
        def _zs2(i, _):
            slice_v[i, pl.ds(0, 16)] = fz
            return 0
        lax.fori_loop(0, SLR, _zs2, 0)


def kernel(x, edge_index, out_index, W_in, b_in, W_out, b_out,
           p0_Wrel, p0_brel, p0_Wroot, p1_Wrel, p1_brel, p1_Wroot):
    N = x.shape[0]
    E = edge_index.shape[1]
    Eo = out_index.shape[1]
    src, dst = edge_index[0], edge_index[1]
    osrc, odst = out_index[0], out_index[1]
    K1, K2 = 5000, 2500
    f32 = x.dtype

    pad, pado = _E2 - E, _E2o - Eo
    dst_p = jnp.concatenate([dst, jnp.full((pad,), _B0 - 1, jnp.int32)])
    osrc_p = jnp.concatenate([osrc, jnp.zeros((pado,), jnp.int32)])
    odst_p = jnp.concatenate([odst, jnp.full((pado,), _Bo - 1, jnp.int32)])

    degh, degoh = _deg_kernel(dst_p, odst_p)

    xw = x @ W_in
    deg = degh[:N] + 1.0
    dinv = lax.rsqrt(deg)
    norm = dinv[src] * dinv[dst]
    s1 = jnp.zeros_like(xw).at[dst].add(xw[src] * norm[:, None])
    h1 = jax.nn.relu(s1 + xw * (dinv * dinv)[:, None] + b_in)

    agg1 = jnp.zeros_like(h1).at[dst].add(h1[src])
    score1 = (agg1 @ p0_Wrel + p0_brel + h1 @ p0_Wroot)[:, 0]
    vals1, perm1 = lax.top_k(score1, K1)
    h2 = h1[perm1] * jnp.tanh(vals1)[:, None]

    mapping = jnp.full((N,), -1, jnp.int32).at[perm1].set(
        jnp.arange(K1, dtype=jnp.int32))
    ns = mapping[src]
    nd = mapping[dst]
    valid = (ns >= 0) & (nd >= 0)
    ns = jnp.where(valid, ns, 0)
    nd = jnp.where(valid, nd, 0)
    em2 = valid.astype(f32)
    agg2 = jnp.zeros((K1, 16), f32).at[nd].add(h2[ns] * em2[:, None])
    score2 = (agg2 @ p1_Wrel + p1_brel + h2 @ p1_Wroot)[:, 0]
    vals2, perm2 = lax.top_k(score2, K2)
    h3 = h2[perm2] * jnp.tanh(vals2)[:, None]

    omask = jnp.ones((Eo,), f32)
    xw3 = h3 @ W_out
    dego = jnp.zeros((K2,), f32).at[odst].add(omask) + 1.0
    dinvo = lax.rsqrt(dego)
    normo = dinvo[osrc] * dinvo[odst] * omask
    s3 = jnp.zeros_like(xw3).at[odst].add(xw3[osrc] * normo[:, None])
    out = jax.nn.relu(s3 + xw3 * (dinvo * dinvo)[:, None] + b_out)
    return out


# speedup vs baseline: 1.0843x; 1.0312x over previous
import functools

import jax
import jax.numpy as jnp
from jax import lax
from jax.experimental import pallas as pl
from jax.experimental.pallas import tpu as pltpu
from jax.experimental.pallas import tpu_sc as plsc

_E2, _B0 = 321536, 10240
_E2o, _Bo = 81920, 2560
_NCH0, _NCHo = 157, 40


def _hist_tile(s, keys_hbm, out_hbm, NCH, B, r):
    SL = B // 16
    keys2d, ones_v, slice_v = r["keys2d"], r["ones_v"], r["slice_v"]
    acc_sh = r["acc_sh"]
    fz = jnp.zeros((16,), jnp.float32)
    tbase = pl.multiple_of(s * (NCH * 128), 8)

    def _zo(i, _):
        ones_v[pl.ds(pl.multiple_of(i * 16, 16), 16)] = fz + 1.0
        return 0

    lax.fori_loop(0, 8, _zo, 0)

    def _zs(i, _):
        slice_v[pl.ds(pl.multiple_of(i * 16, 16), 16)] = fz
        return 0

    lax.fori_loop(0, SL // 16, _zs, 0)
    lo = pl.multiple_of(s * SL, 8)
    pltpu.sync_copy(slice_v.at[pl.ds(0, SL)], acc_sh.at[pl.ds(lo, SL)])
    plsc.subcore_barrier()

    for k in range(NCH):
        pltpu.sync_copy(keys_hbm.at[pl.ds(tbase + k * 128, 128)],
                        keys2d.at[k % 2])
        pltpu.sync_copy(ones_v, acc_sh.at[keys2d.at[k % 2]], add=True)
    plsc.subcore_barrier()

    pltpu.sync_copy(acc_sh.at[pl.ds(lo, SL)], slice_v.at[pl.ds(0, SL)])
    pltpu.sync_copy(slice_v.at[pl.ds(0, SL)], out_hbm.at[pl.ds(lo, SL)])


@functools.partial(
    pl.kernel,
    out_type=(
        jax.ShapeDtypeStruct((_B0,), jnp.float32),
        jax.ShapeDtypeStruct((_Bo,), jnp.float32),
    ),
    mesh=plsc.VectorSubcoreMesh(core_axis_name="c", subcore_axis_name="s"),
    scratch_types=dict(
        keys2d=pltpu.VMEM((2, 128), jnp.int32),
        ones_v=pltpu.VMEM((128,), jnp.float32),
        slice_v=pltpu.VMEM((_B0 // 16,), jnp.float32),
        acc_sh=pltpu.VMEM_SHARED((_B0,), jnp.float32),
    ),
)
def _deg_kernel(dst_f, odst_f, deg_out, dego_out, **r):
    c = lax.axis_index("c")
    s = lax.axis_index("s")

    @pl.when(c == 0)
    def _():
        _hist_tile(s, dst_f, deg_out, _NCH0, _B0, r)

    @pl.when(c == 1)
    def _():
        _hist_tile(s, odst_f, dego_out, _NCHo, _Bo, r)


def kernel(x, edge_index, out_index, W_in, b_in, W_out, b_out,
           p0_Wrel, p0_brel, p0_Wroot, p1_Wrel, p1_brel, p1_Wroot):
    N = x.shape[0]
    E = edge_index.shape[1]
    Eo = out_index.shape[1]
    src, dst = edge_index[0], edge_index[1]
    osrc, odst = out_index[0], out_index[1]
    K1, K2 = 5000, 2500
    f32 = x.dtype

    pad, pado = _E2 - E, _E2o - Eo
    dst_p = jnp.concatenate([dst, jnp.full((pad,), _B0 - 1, jnp.int32)])
    odst_p = jnp.concatenate([odst, jnp.full((pado,), _Bo - 1, jnp.int32)])

    degh, degoh = _deg_kernel(dst_p, odst_p)

    xw = x @ W_in
    deg = degh[:N] + 1.0
    dinv = lax.rsqrt(deg)
    norm = dinv[src] * dinv[dst]
    s1 = jnp.zeros_like(xw).at[dst].add(xw[src] * norm[:, None])
    h1 = jax.nn.relu(s1 + xw * (dinv * dinv)[:, None] + b_in)

    agg1 = jnp.zeros_like(h1).at[dst].add(h1[src])
    score1 = (agg1 @ p0_Wrel + p0_brel + h1 @ p0_Wroot)[:, 0]
    vals1, perm1 = lax.top_k(score1, K1)
    h2 = h1[perm1] * jnp.tanh(vals1)[:, None]

    mapping = jnp.full((N,), -1, jnp.int32).at[perm1].set(
        jnp.arange(K1, dtype=jnp.int32))
    ns = mapping[src]
    nd = mapping[dst]
    valid = (ns >= 0) & (nd >= 0)
    ns = jnp.where(valid, ns, 0)
    nd = jnp.where(valid, nd, 0)
    em2 = valid.astype(f32)
    agg2 = jnp.zeros((K1, 16), f32).at[nd].add(h2[ns] * em2[:, None])
    score2 = (agg2 @ p1_Wrel + p1_brel + h2 @ p1_Wroot)[:, 0]
    vals2, perm2 = lax.top_k(score2, K2)
    h3 = h2[perm2] * jnp.tanh(vals2)[:, None]

    dego = degoh[:K2] + 1.0
    dinvo = lax.rsqrt(dego)
    g3 = h3 * dinvo[:, None]
    s3 = jnp.zeros_like(g3).at[odst].add(g3[osrc])
    out = jax.nn.relu((dinvo[:, None] * (s3 + g3)) @ W_out + b_out)
    return out
